# Initial kernel scaffold; baseline (speedup 1.0000x reference)
import jax
import jax.numpy as jnp
from jax.experimental import pallas as pl


def kernel(input_ids, attention_mask, seq_lengths, adj_matrix, emb, router_W, router_b, gcn1_W, gcn2_W, ln_g, ln_b, syn_W, syn_b, lenS_W, lenS_b, lenL_W, lenL_b, sem_W, sem_b, cls_W, cls_b):
    raise NotImplementedError("write your pallas kernel here")


# speedup vs baseline: 2.2463x; 2.2463x over previous
import functools

import jax
import jax.numpy as jnp
from jax import lax
from jax.experimental import pallas as pl
from jax.experimental.pallas import tpu as pltpu
from jax.experimental.pallas import tpu_sc as plsc

_B, _S, _D = 2, 2048, 1024
_T = _B * _S
_THRESHOLD = 128


_NW = 32
_GCH = 64


def _sc_gather(idx, table):
  t, d = idx.shape[0], table.shape[1]
  b_per_w = t // _NW
  n_ch = b_per_w // _GCH
  mesh = plsc.VectorSubcoreMesh(core_axis_name="c", subcore_axis_name="s")

  @functools.partial(
      pl.kernel,
      mesh=mesh,
      out_type=jax.ShapeDtypeStruct((t, d), jnp.float32),
      scratch_types=[
          pltpu.VMEM((_GCH,), jnp.int32),
          pltpu.VMEM((_GCH, d), jnp.float32),
          pltpu.SemaphoreType.DMA,
      ],
  )
  def k(idx_hbm, table_hbm, out_hbm, idx_v, rows_v, sem):
    wid = lax.axis_index("s") * 2 + lax.axis_index("c")
    base = wid * b_per_w
    for c in range(n_ch):
      off = base + c * _GCH
      pltpu.sync_copy(idx_hbm.at[pl.ds(off, _GCH)], idx_v)
      pltpu.async_copy(table_hbm.at[idx_v], rows_v, sem).wait()
      pltpu.sync_copy(rows_v, out_hbm.at[pl.ds(off, _GCH)])

  return k(idx, table)


_RTB = 512


def _router_body(short_ref, hs_ref, w_ref, b_ref, idx_ref, scl_ref):
  rl = jnp.dot(hs_ref[...], w_ref[...], preferred_element_type=jnp.float32)
  rl = rl + b_ref[...]
  short = short_ref[0, 0, 0] != 0
  neg = jnp.float32(-1e9)
  col = lax.broadcasted_iota(jnp.int32, rl.shape, 1)
  rl = jnp.where(jnp.logical_and(col == 4, short), neg, rl)
  rl = jnp.where(jnp.logical_and(col == 3, jnp.logical_not(short)), neg, rl)
  m = jnp.max(rl, axis=-1, keepdims=True)
  e = jnp.exp(rl - m)
  p = e / jnp.sum(e, axis=-1, keepdims=True)
  syn_p = jnp.max(p[:, 0:3], axis=-1, keepdims=True)
  syn_i = jnp.argmax(p[:, 0:3], axis=-1, keepdims=True)
  len_p = jnp.max(p[:, 3:5], axis=-1, keepdims=True)
  sem_p = jnp.max(p[:, 5:8], axis=-1, keepdims=True)
  sem_i = jnp.argmax(p[:, 5:8], axis=-1, keepdims=True)
  tot = syn_p + len_p + sem_p
  scl = jnp.concatenate([syn_p, len_p, sem_p, tot], axis=-1) / tot
  scl_ref[...] = scl
  idx_ref[...] = jnp.concatenate([syn_i, sem_i], axis=-1).astype(jnp.int32)


def _router(hs2d, router_W, router_b, is_short):
  grid = (_T // _RTB,)
  return pl.pallas_call(
      _router_body,
      grid=grid,
      in_specs=[
          pl.BlockSpec((1, 1, 1), lambda i: (i * _RTB // _S, 0, 0)),
          pl.BlockSpec((_RTB, _D), lambda i: (i, 0)),
          pl.BlockSpec((_D, 8), lambda i: (0, 0)),
          pl.BlockSpec((1, 8), lambda i: (0, 0)),
      ],
      out_specs=[
          pl.BlockSpec((_RTB, 2), lambda i: (i, 0)),
          pl.BlockSpec((_RTB, 4), lambda i: (i, 0)),
      ],
      out_shape=[
          jax.ShapeDtypeStruct((_T, 2), jnp.int32),
          jax.ShapeDtypeStruct((_T, 4), jnp.float32),
      ],
  )(is_short, hs2d, router_W, router_b.reshape(1, 8))


_MMB = 512


def _mm_body(x_ref, w_ref, o_ref):
  o_ref[...] = jnp.dot(x_ref[...], w_ref[...],
                       preferred_element_type=jnp.float32)


def _mm(x, w):
  return pl.pallas_call(
      _mm_body,
      grid=(_T // _MMB,),
      in_specs=[
          pl.BlockSpec((_MMB, _D), lambda i: (i, 0)),
          pl.BlockSpec((_D, _D), lambda i: (0, 0)),
      ],
      out_specs=pl.BlockSpec((_MMB, _D), lambda i: (i, 0)),
      out_shape=jax.ShapeDtypeStruct((_T, _D), jnp.float32),
  )(x, w)


_AMB = 256


def _adj_body(adj_ref, sup_ref, o_ref):
  a = adj_ref[0]
  deg = jnp.maximum(jnp.sum(a, axis=-1, keepdims=True), 1e-9)
  acc = jnp.dot(a, sup_ref[0], preferred_element_type=jnp.float32)
  o_ref[...] = jnp.maximum(acc / deg, 0.0)[None]


def _adj_ln_body(adj_ref, sup_ref, hs_ref, g_ref, b_ref, o_ref):
  a = adj_ref[0]
  deg = jnp.maximum(jnp.sum(a, axis=-1, keepdims=True), 1e-9)
  acc = jnp.dot(a, sup_ref[0], preferred_element_type=jnp.float32)
  x = jnp.maximum(acc / deg, 0.0) + hs_ref[0]
  mu = jnp.mean(x, axis=-1, keepdims=True)
  xc = x - mu
  var = jnp.mean(xc * xc, axis=-1, keepdims=True)
  y = xc * lax.rsqrt(var + 1e-5) * g_ref[...] + b_ref[...]
  o_ref[...] = y[None]


def _adj_mm(adj, sup3d):
  return pl.pallas_call(
      _adj_body,
      grid=(_B, _S // _AMB),
      in_specs=[
          pl.BlockSpec((1, _AMB, _S), lambda b, i: (b, i, 0)),
          pl.BlockSpec((1, _S, _D), lambda b, i: (b, 0, 0)),
      ],
      out_specs=pl.BlockSpec((1, _AMB, _D), lambda b, i: (b, i, 0)),
      out_shape=jax.ShapeDtypeStruct((_B, _S, _D), jnp.float32),
  )(adj, sup3d)


def _adj_mm_ln(adj, sup3d, hs3d, ln_g, ln_b):
  return pl.pallas_call(
      _adj_ln_body,
      grid=(_B, _S // _AMB),
      in_specs=[
          pl.BlockSpec((1, _AMB, _S), lambda b, i: (b, i, 0)),
          pl.BlockSpec((1, _S, _D), lambda b, i: (b, 0, 0)),
          pl.BlockSpec((1, _AMB, _D), lambda b, i: (b, i, 0)),
          pl.BlockSpec((1, _D), lambda b, i: (0, 0)),
          pl.BlockSpec((1, _D), lambda b, i: (0, 0)),
      ],
      out_specs=pl.BlockSpec((1, _AMB, _D), lambda b, i: (b, i, 0)),
      out_shape=jax.ShapeDtypeStruct((_B, _S, _D), jnp.float32),
  )(adj, sup3d, hs3d, ln_g.reshape(1, _D), ln_b.reshape(1, _D))


_XB = 256


def _gelu(x):
  return 0.5 * x * (1.0 + lax.erf(x * 0.7071067811865476))


def _expert_body(sel_ref, idx_ref, scl_ref, sh_ref, hs_ref,
                 synw_ref, synb_ref, lenw_ref, lenb_ref,
                 semw_ref, semb_ref, clsw_ref, clsb_ref, o_ref):
  del sel_ref
  sh = sh_ref[...]
  h = hs_ref[...]
  si = idx_ref[:, 0:1]
  mi = idx_ref[:, 1:2]
  fused = jnp.zeros((_XB, _D), jnp.float32)
  for i in range(3):
    eo = _gelu(jnp.dot(sh, synw_ref[i], preferred_element_type=jnp.float32)
               + synb_ref[i:i + 1, :])
    fused = fused + jnp.where(si == i, scl_ref[:, 0:1], 0.0) * eo
  lo = _gelu(jnp.dot(h, lenw_ref[0], preferred_element_type=jnp.float32)
             + lenb_ref[0])
  fused = fused + scl_ref[:, 1:2] * lo
  for i in range(3):
    eo = _gelu(jnp.dot(h, semw_ref[i], preferred_element_type=jnp.float32)
               + semb_ref[i:i + 1, :])
    fused = fused + jnp.where(mi == i, scl_ref[:, 2:3], 0.0) * eo
  o_ref[...] = (jnp.dot(fused, clsw_ref[...], preferred_element_type=jnp.float32)
                + clsb_ref[...])


def _experts(len_sel, idx, scl, shared2d, hs2d, syn_W, syn_b,
             len_W2, len_b2, sem_W, sem_b, cls_W, cls_b):
  grid_spec = pltpu.PrefetchScalarGridSpec(
      num_scalar_prefetch=1,
      grid=(_T // _XB,),
      in_specs=[
          pl.BlockSpec((_XB, 2), lambda i, sel: (i, 0)),
          pl.BlockSpec((_XB, 4), lambda i, sel: (i, 0)),
          pl.BlockSpec((_XB, _D), lambda i, sel: (i, 0)),
          pl.BlockSpec((_XB, _D), lambda i, sel: (i, 0)),
          pl.BlockSpec((3, _D, _D), lambda i, sel: (0, 0, 0)),
          pl.BlockSpec((3, _D), lambda i, sel: (0, 0)),
          pl.BlockSpec((1, _D, _D), lambda i, sel: (sel[i * _XB // _S], 0, 0)),
          pl.BlockSpec((1, 1, _D), lambda i, sel: (sel[i * _XB // _S], 0, 0)),
          pl.BlockSpec((3, _D, _D), lambda i, sel: (0, 0, 0)),
          pl.BlockSpec((3, _D), lambda i, sel: (0, 0)),
          pl.BlockSpec((_D, 2), lambda i, sel: (0, 0)),
          pl.BlockSpec((1, 2), lambda i, sel: (0, 0)),
      ],
      out_specs=pl.BlockSpec((_XB, 2), lambda i, sel: (i, 0)),
  )
  return pl.pallas_call(
      _expert_body,
      grid_spec=grid_spec,
      out_shape=jax.ShapeDtypeStruct((_T, 2), jnp.float32),
  )(len_sel, idx, scl, shared2d, hs2d, syn_W, syn_b, len_W2, len_b2,
    sem_W, sem_b, cls_W, cls_b.reshape(1, 2))


def kernel(input_ids, attention_mask, seq_lengths, adj_matrix, emb, router_W,
           router_b, gcn1_W, gcn2_W, ln_g, ln_b, syn_W, syn_b, lenS_W, lenS_b,
           lenL_W, lenL_b, sem_W, sem_b, cls_W, cls_b):
  del attention_mask
  ids = input_ids.reshape(_T).astype(jnp.int32)
  hs2d = _sc_gather(ids, emb)
  hs3d = hs2d.reshape(_B, _S, _D)

  is_short = (seq_lengths <= _THRESHOLD)
  idx, scl = _router(hs2d, router_W, router_b,
                     is_short.astype(jnp.int32).reshape(_B, 1, 1))

  t1 = _mm(hs2d, gcn1_W)
  g1 = _adj_mm(adj_matrix, t1.reshape(_B, _S, _D))
  t2 = _mm(g1.reshape(_T, _D), gcn2_W)
  shared = _adj_mm_ln(adj_matrix, t2.reshape(_B, _S, _D), hs3d, ln_g, ln_b)

  len_sel = jnp.where(is_short, 0, 1).astype(jnp.int32)
  len_W2 = jnp.stack([lenS_W, lenL_W])
  len_b2 = jnp.stack([lenS_b, lenL_b]).reshape(2, 1, _D)
  logits = _experts(len_sel, idx, scl, shared.reshape(_T, _D), hs2d,
                    syn_W, syn_b, len_W2, len_b2, sem_W, sem_b, cls_W, cls_b)
  return logits.reshape(_B, _S, 2)


# speedup vs baseline: 2.2674x; 1.0094x over previous
import functools

import jax
import jax.numpy as jnp
from jax import lax
from jax.experimental import pallas as pl
from jax.experimental.pallas import tpu as pltpu
from jax.experimental.pallas import tpu_sc as plsc

_B, _S, _D = 2, 2048, 1024
_T = _B * _S
_THRESHOLD = 128


_NW = 32
_GCH = 64


def _sc_gather(idx, table):
  t, d = idx.shape[0], table.shape[1]
  b_per_w = t // _NW
  n_ch = b_per_w // _GCH
  mesh = plsc.VectorSubcoreMesh(core_axis_name="c", subcore_axis_name="s")

  @functools.partial(
      pl.kernel,
      mesh=mesh,
      out_type=jax.ShapeDtypeStruct((t, d), jnp.float32),
      scratch_types=[
          pltpu.VMEM((_GCH,), jnp.int32),
          pltpu.VMEM((_GCH, d), jnp.float32),
          pltpu.SemaphoreType.DMA,
      ],
  )
  def k(idx_hbm, table_hbm, out_hbm, idx_v, rows_v, sem):
    wid = lax.axis_index("s") * 2 + lax.axis_index("c")
    base = wid * b_per_w
    for c in range(n_ch):
      off = base + c * _GCH
      pltpu.sync_copy(idx_hbm.at[pl.ds(off, _GCH)], idx_v)
      pltpu.async_copy(table_hbm.at[idx_v], rows_v, sem).wait()
      pltpu.sync_copy(rows_v, out_hbm.at[pl.ds(off, _GCH)])

  return k(idx, table)


_RTB = 512


def _router_body(short_ref, hs_ref, w_ref, b_ref, idx_ref, scl_ref):
  rl = jnp.dot(hs_ref[...], w_ref[...], preferred_element_type=jnp.float32)
  rl = rl + b_ref[...]
  short = short_ref[0, 0, 0] != 0
  neg = jnp.float32(-1e9)
  col = lax.broadcasted_iota(jnp.int32, rl.shape, 1)
  rl = jnp.where(jnp.logical_and(col == 4, short), neg, rl)
  rl = jnp.where(jnp.logical_and(col == 3, jnp.logical_not(short)), neg, rl)
  m = jnp.max(rl, axis=-1, keepdims=True)
  e = jnp.exp(rl - m)
  p = e / jnp.sum(e, axis=-1, keepdims=True)
  syn_p = jnp.max(p[:, 0:3], axis=-1, keepdims=True)
  syn_i = jnp.argmax(p[:, 0:3], axis=-1, keepdims=True)
  len_p = jnp.max(p[:, 3:5], axis=-1, keepdims=True)
  sem_p = jnp.max(p[:, 5:8], axis=-1, keepdims=True)
  sem_i = jnp.argmax(p[:, 5:8], axis=-1, keepdims=True)
  tot = syn_p + len_p + sem_p
  scl = jnp.concatenate([syn_p, len_p, sem_p, tot], axis=-1) / tot
  scl_ref[...] = scl
  idx_ref[...] = jnp.concatenate([syn_i, sem_i], axis=-1).astype(jnp.int32)


def _router(hs2d, router_W, router_b, is_short):
  grid = (_T // _RTB,)
  return pl.pallas_call(
      _router_body,
      grid=grid,
      in_specs=[
          pl.BlockSpec((1, 1, 1), lambda i: (i * _RTB // _S, 0, 0)),
          pl.BlockSpec((_RTB, _D), lambda i: (i, 0)),
          pl.BlockSpec((_D, 8), lambda i: (0, 0)),
          pl.BlockSpec((1, 8), lambda i: (0, 0)),
      ],
      out_specs=[
          pl.BlockSpec((_RTB, 2), lambda i: (i, 0)),
          pl.BlockSpec((_RTB, 4), lambda i: (i, 0)),
      ],
      out_shape=[
          jax.ShapeDtypeStruct((_T, 2), jnp.int32),
          jax.ShapeDtypeStruct((_T, 4), jnp.float32),
      ],
  )(is_short, hs2d, router_W, router_b.reshape(1, 8))


_MMB = 512


def _mm_body(x_ref, w_ref, o_ref):
  o_ref[...] = jnp.dot(x_ref[...], w_ref[...],
                       preferred_element_type=jnp.float32)


def _mm(x, w):
  return pl.pallas_call(
      _mm_body,
      grid=(_T // _MMB,),
      in_specs=[
          pl.BlockSpec((_MMB, _D), lambda i: (i, 0)),
          pl.BlockSpec((_D, _D), lambda i: (0, 0)),
      ],
      out_specs=pl.BlockSpec((_MMB, _D), lambda i: (i, 0)),
      out_shape=jax.ShapeDtypeStruct((_T, _D), jnp.float32),
  )(x, w)


_AMB = 256


def _adj_body(adj_ref, sup_ref, o_ref):
  a = adj_ref[0]
  deg = jnp.maximum(jnp.sum(a, axis=-1, keepdims=True), 1e-9)
  acc = jnp.dot(a, sup_ref[0], preferred_element_type=jnp.float32)
  o_ref[...] = jnp.maximum(acc / deg, 0.0)[None]


def _adj_ln_body(adj_ref, sup_ref, hs_ref, g_ref, b_ref, o_ref):
  a = adj_ref[0]
  deg = jnp.maximum(jnp.sum(a, axis=-1, keepdims=True), 1e-9)
  acc = jnp.dot(a, sup_ref[0], preferred_element_type=jnp.float32)
  x = jnp.maximum(acc / deg, 0.0) + hs_ref[0]
  mu = jnp.mean(x, axis=-1, keepdims=True)
  xc = x - mu
  var = jnp.mean(xc * xc, axis=-1, keepdims=True)
  y = xc * lax.rsqrt(var + 1e-5) * g_ref[...] + b_ref[...]
  o_ref[...] = y[None]


def _adj_mm(adj, sup3d):
  return pl.pallas_call(
      _adj_body,
      grid=(_B, _S // _AMB),
      in_specs=[
          pl.BlockSpec((1, _AMB, _S), lambda b, i: (b, i, 0)),
          pl.BlockSpec((1, _S, _D), lambda b, i: (b, 0, 0)),
      ],
      out_specs=pl.BlockSpec((1, _AMB, _D), lambda b, i: (b, i, 0)),
      out_shape=jax.ShapeDtypeStruct((_B, _S, _D), jnp.float32),
  )(adj, sup3d)


def _adj_mm_ln(adj, sup3d, hs3d, ln_g, ln_b):
  return pl.pallas_call(
      _adj_ln_body,
      grid=(_B, _S // _AMB),
      in_specs=[
          pl.BlockSpec((1, _AMB, _S), lambda b, i: (b, i, 0)),
          pl.BlockSpec((1, _S, _D), lambda b, i: (b, 0, 0)),
          pl.BlockSpec((1, _AMB, _D), lambda b, i: (b, i, 0)),
          pl.BlockSpec((1, _D), lambda b, i: (0, 0)),
          pl.BlockSpec((1, _D), lambda b, i: (0, 0)),
      ],
      out_specs=pl.BlockSpec((1, _AMB, _D), lambda b, i: (b, i, 0)),
      out_shape=jax.ShapeDtypeStruct((_B, _S, _D), jnp.float32),
  )(adj, sup3d, hs3d, ln_g.reshape(1, _D), ln_b.reshape(1, _D))


_XB = 256


def _gelu(x):
  return 0.5 * x * (1.0 + lax.erf(x * 0.7071067811865476))


def _expert_body(sel_ref, idx_ref, scl_ref, sh_ref, hs_ref,
                 synw_ref, synb_ref, lenw_ref, lenb_ref,
                 semw_ref, semb_ref, clsw_ref, clsb_ref, o_ref):
  del sel_ref
  sh = sh_ref[...].astype(jnp.bfloat16)
  h = hs_ref[...].astype(jnp.bfloat16)
  si = idx_ref[:, 0:1]
  mi = idx_ref[:, 1:2]
  fused = jnp.zeros((_XB, _D), jnp.float32)
  for i in range(3):
    eo = _gelu(jnp.dot(sh, synw_ref[i].astype(jnp.bfloat16),
                       preferred_element_type=jnp.float32)
               + synb_ref[i:i + 1, :])
    fused = fused + jnp.where(si == i, scl_ref[:, 0:1], 0.0) * eo
  lo = _gelu(jnp.dot(h, lenw_ref[0].astype(jnp.bfloat16),
                     preferred_element_type=jnp.float32)
             + lenb_ref[0])
  fused = fused + scl_ref[:, 1:2] * lo
  for i in range(3):
    eo = _gelu(jnp.dot(h, semw_ref[i].astype(jnp.bfloat16),
                       preferred_element_type=jnp.float32)
               + semb_ref[i:i + 1, :])
    fused = fused + jnp.where(mi == i, scl_ref[:, 2:3], 0.0) * eo
  o_ref[...] = (jnp.dot(fused, clsw_ref[...], preferred_element_type=jnp.float32)
                + clsb_ref[...])


def _experts(len_sel, idx, scl, shared2d, hs2d, syn_W, syn_b,
             len_W2, len_b2, sem_W, sem_b, cls_W, cls_b):
  grid_spec = pltpu.PrefetchScalarGridSpec(
      num_scalar_prefetch=1,
      grid=(_T // _XB,),
      in_specs=[
          pl.BlockSpec((_XB, 2), lambda i, sel: (i, 0)),
          pl.BlockSpec((_XB, 4), lambda i, sel: (i, 0)),
          pl.BlockSpec((_XB, _D), lambda i, sel: (i, 0)),
          pl.BlockSpec((_XB, _D), lambda i, sel: (i, 0)),
          pl.BlockSpec((3, _D, _D), lambda i, sel: (0, 0, 0)),
          pl.BlockSpec((3, _D), lambda i, sel: (0, 0)),
          pl.BlockSpec((1, _D, _D), lambda i, sel: (sel[i * _XB // _S], 0, 0)),
          pl.BlockSpec((1, 1, _D), lambda i, sel: (sel[i * _XB // _S], 0, 0)),
          pl.BlockSpec((3, _D, _D), lambda i, sel: (0, 0, 0)),
          pl.BlockSpec((3, _D), lambda i, sel: (0, 0)),
          pl.BlockSpec((_D, 2), lambda i, sel: (0, 0)),
          pl.BlockSpec((1, 2), lambda i, sel: (0, 0)),
      ],
      out_specs=pl.BlockSpec((_XB, 2), lambda i, sel: (i, 0)),
  )
  return pl.pallas_call(
      _expert_body,
      grid_spec=grid_spec,
      out_shape=jax.ShapeDtypeStruct((_T, 2), jnp.float32),
  )(len_sel, idx, scl, shared2d, hs2d, syn_W, syn_b, len_W2, len_b2,
    sem_W, sem_b, cls_W, cls_b.reshape(1, 2))


def kernel(input_ids, attention_mask, seq_lengths, adj_matrix, emb, router_W,
           router_b, gcn1_W, gcn2_W, ln_g, ln_b, syn_W, syn_b, lenS_W, lenS_b,
           lenL_W, lenL_b, sem_W, sem_b, cls_W, cls_b):
  del attention_mask
  ids = input_ids.reshape(_T).astype(jnp.int32)
  hs2d = _sc_gather(ids, emb)
  hs3d = hs2d.reshape(_B, _S, _D)

  is_short = (seq_lengths <= _THRESHOLD)
  idx, scl = _router(hs2d, router_W, router_b,
                     is_short.astype(jnp.int32).reshape(_B, 1, 1))

  t1 = _mm(hs2d, gcn1_W)
  g1 = _adj_mm(adj_matrix, t1.reshape(_B, _S, _D))
  t2 = _mm(g1.reshape(_T, _D), gcn2_W)
  shared = _adj_mm_ln(adj_matrix, t2.reshape(_B, _S, _D), hs3d, ln_g, ln_b)

  len_sel = jnp.where(is_short, 0, 1).astype(jnp.int32)
  len_W2 = jnp.stack([lenS_W, lenL_W])
  len_b2 = jnp.stack([lenS_b, lenL_b]).reshape(2, 1, _D)
  logits = _experts(len_sel, idx, scl, shared.reshape(_T, _D), hs2d,
                    syn_W, syn_b, len_W2, len_b2, sem_W, sem_b, cls_W, cls_b)
  return logits.reshape(_B, _S, 2)
